# Initial kernel scaffold; baseline (speedup 1.0000x reference)
import jax
import jax.numpy as jnp
from jax.experimental import pallas as pl


def kernel(feature, edge_index, edge_type, W1, a_src1, a_dst1, b1, W2, a_src2, a_dst2, b2, W3, a_src3, a_dst3, b3, W4, a_src4, a_dst4, b4):
    raise NotImplementedError("write your pallas kernel here")


# speedup vs baseline: 28.1054x; 28.1054x over previous
import functools

import jax
import jax.numpy as jnp
from jax import lax
from jax.experimental import pallas as pl
from jax.experimental.pallas import tpu as pltpu
from jax.experimental.pallas import tpu_sc as plsc

NW = 32
CH = 512
BLK = 400

f32 = jnp.float32


def _rup(x, m):
    return (x + m - 1) // m * m


def _tc_first(x, W, AW):
    Nn, K = x.shape
    H2 = AW.shape[1]
    nb = Nn // BLK

    def body(x_ref, w_ref, a_ref, h_ref, al_ref, am_ref, mx):
        i = pl.program_id(0)
        h = jnp.dot(x_ref[...], w_ref[...], preferred_element_type=f32)
        al = jnp.dot(h, a_ref[...], preferred_element_type=f32)
        h_ref[...] = h
        al_ref[...] = al
        m = jnp.broadcast_to(jnp.max(al, axis=0, keepdims=True), (8, H2))

        @pl.when(i == 0)
        def _():
            mx[...] = m

        @pl.when(i > 0)
        def _():
            mx[...] = jnp.maximum(mx[...], m)

        am_ref[...] = mx[...]

    return pl.pallas_call(
        body,
        grid=(nb,),
        in_specs=[pl.BlockSpec((BLK, K), lambda i: (i, 0)),
                  pl.BlockSpec((K, 128), lambda i: (0, 0)),
                  pl.BlockSpec((128, H2), lambda i: (0, 0))],
        out_specs=[pl.BlockSpec((BLK, 128), lambda i: (i, 0)),
                   pl.BlockSpec((BLK, H2), lambda i: (i, 0)),
                   pl.BlockSpec((8, H2), lambda i: (0, 0))],
        out_shape=[jax.ShapeDtypeStruct((Nn, 128), f32),
                   jax.ShapeDtypeStruct((Nn, H2), f32),
                   jax.ShapeDtypeStruct((8, H2), f32)],
        scratch_shapes=[pltpu.VMEM((8, H2), f32)],
    )(x, W, AW)


def _tc_mid(p0, p1, b, W, AW):
    Nn = p0.shape[0]
    H2 = AW.shape[1]
    nb = Nn // BLK

    def body(p0_ref, p1_ref, b_ref, w_ref, a_ref, h_ref, al_ref, am_ref, mx):
        i = pl.program_id(0)
        x = p0_ref[...] + p1_ref[...] + b_ref[...]
        h = jnp.dot(x, w_ref[...], preferred_element_type=f32)
        al = jnp.dot(h, a_ref[...], preferred_element_type=f32)
        h_ref[...] = h
        al_ref[...] = al
        m = jnp.broadcast_to(jnp.max(al, axis=0, keepdims=True), (8, H2))

        @pl.when(i == 0)
        def _():
            mx[...] = m

        @pl.when(i > 0)
        def _():
            mx[...] = jnp.maximum(mx[...], m)

        am_ref[...] = mx[...]

    return pl.pallas_call(
        body,
        grid=(nb,),
        in_specs=[pl.BlockSpec((BLK, 128), lambda i: (i, 0)),
                  pl.BlockSpec((BLK, 128), lambda i: (i, 0)),
                  pl.BlockSpec((1, 128), lambda i: (0, 0)),
                  pl.BlockSpec((128, 128), lambda i: (0, 0)),
                  pl.BlockSpec((128, H2), lambda i: (0, 0))],
        out_specs=[pl.BlockSpec((BLK, 128), lambda i: (i, 0)),
                   pl.BlockSpec((BLK, H2), lambda i: (i, 0)),
                   pl.BlockSpec((8, H2), lambda i: (0, 0))],
        out_shape=[jax.ShapeDtypeStruct((Nn, 128), f32),
                   jax.ShapeDtypeStruct((Nn, H2), f32),
                   jax.ShapeDtypeStruct((8, H2), f32)],
        scratch_shapes=[pltpu.VMEM((8, H2), f32)],
    )(p0, p1, b, W, AW)


def _tc_final(q10, q11, b3, q20, q21, b4):
    Nn = q10.shape[0]
    nb = Nn // BLK

    def body(a_ref, b_ref, bb3, c_ref, d_ref, bb4, o_ref):
        o_ref[...] = (a_ref[...] + b_ref[...] + bb3[...]
                      + c_ref[...] + d_ref[...] + bb4[...])

    row = pl.BlockSpec((BLK, 128), lambda i: (i, 0))
    one = pl.BlockSpec((1, 128), lambda i: (0, 0))
    return pl.pallas_call(
        body,
        grid=(nb,),
        in_specs=[row, row, one, row, row, one],
        out_specs=row,
        out_shape=jax.ShapeDtypeStruct((Nn, 128), f32),
    )(q10, q11, b3, q20, q21, b4)


def _sc_pass_a(alog1d, amaxm, srcp, dstp, Nn, H):
    H2 = 2 * H
    E_PAD = srcp.shape[0]
    EPW = E_PAD // NW
    NPAD = _rup(Nn + 16, 256)
    NH = NPAD * H
    KB = CH * H // 128
    Z = NH // 16
    nchunks = EPW // CH
    mesh = plsc.VectorSubcoreMesh(core_axis_name="c", subcore_axis_name="s", num_cores=2, num_subcores=16)

    @functools.partial(
        pl.kernel,
        mesh=mesh,
        compiler_params=pltpu.CompilerParams(needs_layout_passes=False),
        out_type=[jax.ShapeDtypeStruct((H, E_PAD), f32),
                  jax.ShapeDtypeStruct((2, NH), f32)],
        scratch_types=[pltpu.VMEM(((Nn + 16) * H2,), f32),
                       pltpu.VMEM((16,), f32),
                       pltpu.VMEM((CH,), jnp.int32),
                       pltpu.VMEM((CH,), jnp.int32),
                       pltpu.VMEM((H, CH), f32),
                       pltpu.VMEM((KB, 128), jnp.int32),
                       pltpu.VMEM((KB, 128), f32),
                       pltpu.VMEM((Z,), f32),
                       pltpu.VMEM_SHARED((NH,), f32)],
    )
    def k(alog_h, amax_h, src_h, dst_h, ex_h, den_h,
          alog_t, amax_t, src_b, dst_b, exb, sidx, sval, zb, den_sh):
        c = lax.axis_index("c")
        s = lax.axis_index("s")
        wid = c * 16 + s
        zeros16 = jnp.zeros((16,), f32)
        pltpu.sync_copy(alog_h, alog_t.at[pl.ds(0, Nn * H2)])
        for kk in range(H2):
            alog_t[pl.ds(Nn * H2 + kk * 16, 16)] = zeros16
        pltpu.sync_copy(amax_h.at[pl.ds(0, 16)], amax_t)

        def zbody(i, _):
            zb[pl.ds(i * 16, 16)] = zeros16
            return 0
        lax.fori_loop(0, Z // 16, zbody, 0)
        pltpu.sync_copy(zb, den_sh.at[pl.ds(s * Z, Z)])
        plsc.subcore_barrier()

        def chunk(i, _):
            base = wid * EPW + i * CH
            pltpu.sync_copy(src_h.at[pl.ds(base, CH)], src_b)
            pltpu.sync_copy(dst_h.at[pl.ds(base, CH)], dst_b)

            def grp(g, _):
                s16 = src_b[pl.ds(g * 16, 16)]
                d16 = dst_b[pl.ds(g * 16, 16)]
                amv = amax_t[pl.ds(0, 16)]
                for h in range(H):
                    asv = plsc.load_gather(alog_t, [s16 * H2 + h])
                    adv = plsc.load_gather(alog_t, [d16 * H2 + (H + h)])
                    t = asv + adv
                    al = jnp.where(t >= 0, t, 0.2 * t)
                    bb = amv[h] + adv
                    bl = jnp.where(bb >= 0, bb, 0.2 * bb)
                    ex = jnp.exp(al - bl)
                    exb[h, pl.ds(g * 16, 16)] = ex
                    flat = (g * H + h) * 16
                    row = flat // 128
                    col = flat % 128
                    sidx[row, pl.ds(col, 16)] = d16 * H + h
                    sval[row, pl.ds(col, 16)] = ex
                return 0
            lax.fori_loop(0, CH // 16, grp, 0)

            for h in range(H):
                pltpu.sync_copy(exb.at[h], ex_h.at[h, pl.ds(base, CH)])
            for j in range(KB):
                pltpu.sync_copy(sval.at[j], den_sh.at[sidx.at[j]], add=True)
            return 0
        lax.fori_loop(0, nchunks, chunk, 0)
        plsc.subcore_barrier()
        pltpu.sync_copy(den_sh.at[pl.ds(s * Z, Z)], den_h.at[c, pl.ds(s * Z, Z)])

    return k(alog1d, amaxm, srcp, dstp)


def _sc_pass_b(srcp, dstp, exh, denh, harr, Nn, H):
    C = 128 // H
    CHB = 128
    E_PAD = srcp.shape[0]
    EPW = E_PAD // NW
    NPAD = _rup(Nn + 16, 256)
    NH = NPAD * H
    NHS = NH // 16
    nchunks = EPW // CHB
    rpt = NPAD // 16
    mesh = plsc.VectorSubcoreMesh(core_axis_name="c", subcore_axis_name="s", num_cores=2, num_subcores=16)

    @functools.partial(
        pl.kernel,
        mesh=mesh,
        compiler_params=pltpu.CompilerParams(needs_layout_passes=False),
        out_type=jax.ShapeDtypeStruct((2, NPAD, 128), f32),
        scratch_types=[pltpu.VMEM((CHB * H,), f32),
                       pltpu.VMEM((CHB * H,), f32),
                       pltpu.VMEM((CHB * H,), f32),
                       pltpu.VMEM((1, 128), jnp.int32),
                       pltpu.VMEM((1, 128), jnp.int32),
                       pltpu.VMEM((H, CHB), f32),
                       pltpu.VMEM((CHB, 128), f32),
                       pltpu.VMEM_SHARED((NPAD, 128), f32),
                       pltpu.SemaphoreType.DMA],
    )
    def k(src_h, dst_h, ex_h, den_h, hm_h, outp,
          t0, t1, rloc, srcb, dstb, exb, rows, out_acc, sem):
        c = lax.axis_index("c")
        s = lax.axis_index("s")
        wid = c * 16 + s
        zeros16 = jnp.zeros((16,), f32)

        def zr(i, _):
            rows[i // 8, pl.ds((i % 8) * 16, 16)] = zeros16
            return 0
        lax.fori_loop(0, CHB * 8, zr, 0)
        base0 = s * rpt
        for j in range(rpt // CHB):
            pltpu.sync_copy(rows, out_acc.at[pl.ds(base0 + j * CHB, CHB)])
        rem = rpt % CHB
        if rem:
            pltpu.sync_copy(rows.at[pl.ds(0, rem)],
                            out_acc.at[pl.ds(base0 + rpt - rem, rem)])
        plsc.subcore_barrier()

        def chunk(i, _):
            base = wid * EPW + i * CHB
            pltpu.sync_copy(src_h.at[pl.ds(base, CHB)], srcb.at[0])
            pltpu.sync_copy(dst_h.at[pl.ds(base, CHB)], dstb.at[0])
            for h in range(H):
                pltpu.sync_copy(ex_h.at[h, pl.ds(base, CHB)], exb.at[h])
            cp = pltpu.async_copy(hm_h.at[srcb.at[0]], rows, sem)
            cp.wait()

            def mulg(g, _):
                cv = [exb[h, pl.ds(g * 16, 16)] for h in range(H)]
                for l in range(16):
                    e = g * 16 + l
                    for h in range(H):
                        cs = cv[h][l]
                        for v in range(C // 16):
                            sl = pl.ds(h * C + v * 16, 16)
                            rows[e, sl] = rows[e, sl] * cs
                return 0
            lax.fori_loop(0, CHB // 16, mulg, 0)

            pltpu.sync_copy(rows, out_acc.at[dstb.at[0]], add=True)
            return 0
        lax.fori_loop(0, nchunks, chunk, 0)
        plsc.subcore_barrier()

        RPG = 16 // H
        for j in range(rpt // CHB):
            r0 = s * rpt + j * CHB
            pltpu.sync_copy(out_acc.at[pl.ds(r0, CHB)], rows)
            pltpu.sync_copy(den_h.at[0, pl.ds(r0 * H, CHB * H)], t0)
            pltpu.sync_copy(den_h.at[1, pl.ds(r0 * H, CHB * H)], t1)

            def rb(i, _):
                rloc[pl.ds(i * 16, 16)] = 1.0 / (
                    t0[pl.ds(i * 16, 16)] + t1[pl.ds(i * 16, 16)] + 1e-16)
                return 0
            lax.fori_loop(0, CHB * H // 16, rb, 0)

            def sg(g, _):
                dv = rloc[pl.ds(g * 16, 16)]
                for l in range(RPG):
                    e = g * RPG + l
                    for h in range(H):
                        cs = dv[l * H + h]
                        for v in range(C // 16):
                            sl = pl.ds(h * C + v * 16, 16)
                            rows[e, sl] = rows[e, sl] * cs
                return 0
            lax.fori_loop(0, CHB * H // 16, sg, 0)
            pltpu.sync_copy(rows, outp.at[c, pl.ds(r0, CHB)])

    return k(srcp, dstp, exh, denh, harr)


def _mk_aw(a_s, a_d):
    Hh, Cc = a_s.shape
    eye = jnp.eye(Hh, dtype=f32)
    Ms = (a_s[:, :, None] * eye[:, None, :]).reshape(Hh * Cc, Hh)
    Md = (a_d[:, :, None] * eye[:, None, :]).reshape(Hh * Cc, Hh)
    return jnp.concatenate([Ms, Md], axis=1)


def kernel(feature, edge_index, edge_type,
           W1, a_src1, a_dst1, b1,
           W2, a_src2, a_dst2, b2,
           W3, a_src3, a_dst3, b3,
           W4, a_src4, a_dst4, b4):
    Nn = feature.shape[0]
    E = edge_index.shape[1]
    E_PAD = _rup(E, NW * CH)

    feat1 = feature[:, :1536]
    feat2 = jnp.concatenate([feature[:, :768], feature[:, 1536:]], axis=1)
    srcp = jnp.concatenate(
        [edge_index[0], jnp.zeros((E_PAD - E,), jnp.int32)])
    dstp = jnp.concatenate(
        [edge_index[1], jnp.full((E_PAD - E,), Nn, jnp.int32)])

    def gat_first(x, W, a_s, a_d):
        h, alog, amax = _tc_first(x, W, _mk_aw(a_s, a_d))
        ex, den = _sc_pass_a(alog.reshape(-1), amax.reshape(-1), srcp, dstp,
                             Nn, a_s.shape[0])
        return _sc_pass_b(srcp, dstp, ex, den, h, Nn, a_s.shape[0])[:, :Nn]

    def gat_mid(p, bias, W, a_s, a_d):
        h, alog, amax = _tc_mid(p[0], p[1], bias.reshape(1, -1), W,
                                _mk_aw(a_s, a_d))
        ex, den = _sc_pass_a(alog.reshape(-1), amax.reshape(-1), srcp, dstp,
                             Nn, a_s.shape[0])
        return _sc_pass_b(srcp, dstp, ex, den, h, Nn, a_s.shape[0])[:, :Nn]

    P1 = gat_first(feat1, W1, a_src1, a_dst1)
    Q1 = gat_mid(P1, b1, W3, a_src3, a_dst3)
    P2 = gat_first(feat2, W2, a_src2, a_dst2)
    Q2 = gat_mid(P2, b2, W4, a_src4, a_dst4)
    return _tc_final(Q1[0], Q1[1], b3.reshape(1, -1),
                     Q2[0], Q2[1], b4.reshape(1, -1))
